# baseline (device time: 32389 ns/iter reference)
import jax
import jax.numpy as jnp
from jax import lax
from jax.experimental import pallas as pl
from jax.experimental.pallas import tpu as pltpu

N_DEV = 4


def kernel(A, B):
    M, _ = A.shape
    _, N = B.shape
    Mb = M // N_DEV

    A = A.astype(jnp.bfloat16)
    B = B.astype(jnp.bfloat16)

    comm_dtype = jnp.float32

    def body(a_ref, b_ref, out_ref, send_buf, recv_buf, send_sems, recv_sems):
        my = lax.axis_index("i")

        barrier = pltpu.get_barrier_semaphore()
        for d in range(1, N_DEV):
            pl.semaphore_signal(
                barrier,
                inc=1,
                device_id=((my + d) % N_DEV,),
                device_id_type=pl.DeviceIdType.MESH,
            )
        pl.semaphore_wait(barrier, N_DEV - 1)

        rdmas = []
        for d in range(1, N_DEV):
            q = (my + d) % N_DEV
            a_blk = a_ref[pl.ds(q * Mb, Mb), :]
            p_blk = jnp.dot(a_blk, b_ref[:, :], preferred_element_type=jnp.float32)
            send_buf[d - 1, :, :] = p_blk.astype(comm_dtype)
            rdma = pltpu.make_async_remote_copy(
                src_ref=send_buf.at[d - 1],
                dst_ref=recv_buf.at[d - 1],
                send_sem=send_sems.at[d - 1],
                recv_sem=recv_sems.at[d - 1],
                device_id=(q,),
                device_id_type=pl.DeviceIdType.MESH,
            )
            rdma.start()
            rdmas.append(rdma)

        own = jnp.dot(
            a_ref[pl.ds(my * Mb, Mb), :],
            b_ref[:, :],
            preferred_element_type=jnp.float32,
        )

        for rdma in rdmas:
            rdma.wait_recv()
        acc = own
        for j in range(N_DEV - 1):
            acc = acc + recv_buf[j, :, :].astype(jnp.float32)
        out_ref[:, :] = acc

        for rdma in rdmas:
            rdma.wait_send()

    return pl.pallas_call(
        body,
        out_shape=jax.ShapeDtypeStruct((Mb, N), jnp.float32),
        in_specs=[
            pl.BlockSpec(memory_space=pltpu.VMEM),
            pl.BlockSpec(memory_space=pltpu.VMEM),
        ],
        out_specs=pl.BlockSpec(memory_space=pltpu.VMEM),
        scratch_shapes=[
            pltpu.VMEM((N_DEV - 1, Mb, N), comm_dtype),
            pltpu.VMEM((N_DEV - 1, Mb, N), comm_dtype),
            pltpu.SemaphoreType.DMA((N_DEV - 1,)),
            pltpu.SemaphoreType.DMA((N_DEV - 1,)),
        ],
        compiler_params=pltpu.CompilerParams(collective_id=0),
    )(A, B)


# device time: 21207 ns/iter; 1.5273x vs baseline; 1.5273x over previous
import jax
import jax.numpy as jnp
from jax import lax
from jax.experimental import pallas as pl
from jax.experimental.pallas import tpu as pltpu

N_DEV = 4


def kernel(A, B):
    M, _ = A.shape
    _, N = B.shape
    Mb = M // N_DEV

    A = A.astype(jnp.bfloat16)
    B = B.astype(jnp.bfloat16)

    comm_dtype = jnp.bfloat16

    def body(a_ref, b_ref, out_ref, send_buf, recv_buf, send_sems, recv_sems):
        my = lax.axis_index("i")

        barrier = pltpu.get_barrier_semaphore()
        for d in range(1, N_DEV):
            pl.semaphore_signal(
                barrier,
                inc=1,
                device_id=((my + d) % N_DEV,),
                device_id_type=pl.DeviceIdType.MESH,
            )
        pl.semaphore_wait(barrier, N_DEV - 1)

        rdmas = []
        for d in range(1, N_DEV):
            q = (my + d) % N_DEV
            a_blk = a_ref[pl.ds(q * Mb, Mb), :]
            p_blk = jnp.dot(a_blk, b_ref[:, :], preferred_element_type=jnp.float32)
            send_buf[d - 1, :, :] = p_blk.astype(comm_dtype)
            rdma = pltpu.make_async_remote_copy(
                src_ref=send_buf.at[d - 1],
                dst_ref=recv_buf.at[d - 1],
                send_sem=send_sems.at[d - 1],
                recv_sem=recv_sems.at[d - 1],
                device_id=(q,),
                device_id_type=pl.DeviceIdType.MESH,
            )
            rdma.start()
            rdmas.append(rdma)

        own = jnp.dot(
            a_ref[pl.ds(my * Mb, Mb), :],
            b_ref[:, :],
            preferred_element_type=jnp.float32,
        )

        for rdma in rdmas:
            rdma.wait_recv()
        acc = own
        for j in range(N_DEV - 1):
            acc = acc + recv_buf[j, :, :].astype(jnp.float32)
        out_ref[:, :] = acc

        for rdma in rdmas:
            rdma.wait_send()

    return pl.pallas_call(
        body,
        out_shape=jax.ShapeDtypeStruct((Mb, N), jnp.float32),
        in_specs=[
            pl.BlockSpec(memory_space=pltpu.VMEM),
            pl.BlockSpec(memory_space=pltpu.VMEM),
        ],
        out_specs=pl.BlockSpec(memory_space=pltpu.VMEM),
        scratch_shapes=[
            pltpu.VMEM((N_DEV - 1, Mb, N), comm_dtype),
            pltpu.VMEM((N_DEV - 1, Mb, N), comm_dtype),
            pltpu.SemaphoreType.DMA((N_DEV - 1,)),
            pltpu.SemaphoreType.DMA((N_DEV - 1,)),
        ],
        compiler_params=pltpu.CompilerParams(collective_id=0),
    )(A, B)


# device time: 20601 ns/iter; 1.5722x vs baseline; 1.0294x over previous
import jax
import jax.numpy as jnp
from jax import lax
from jax.experimental import pallas as pl
from jax.experimental.pallas import tpu as pltpu

N_DEV = 4


def kernel(A, B):
    M, _ = A.shape
    _, N = B.shape
    Mb = M // N_DEV

    A = A.astype(jnp.bfloat16)
    B = B.astype(jnp.bfloat16)

    comm_dtype = jnp.bfloat16

    def body(a_ref, b_ref, out_ref, send_buf, recv_buf, send_sems, recv_sems):
        my = lax.axis_index("i")

        barrier = pltpu.get_barrier_semaphore()
        for d in range(1, N_DEV):
            pl.semaphore_signal(
                barrier,
                inc=1,
                device_id=((my + d) % N_DEV,),
                device_id_type=pl.DeviceIdType.MESH,
            )
        pl.semaphore_wait(barrier, N_DEV - 1)

        rdmas = []
        for d in range(1, N_DEV):
            q = (my + d) % N_DEV
            a_blk = a_ref[pl.ds(q * Mb, Mb), :]
            send_buf[d - 1, :, :] = jnp.dot(
                a_blk, b_ref[:, :], preferred_element_type=jnp.float32
            ).astype(comm_dtype)
            rdma = pltpu.make_async_remote_copy(
                src_ref=send_buf.at[d - 1],
                dst_ref=recv_buf.at[d - 1],
                send_sem=send_sems.at[d - 1],
                recv_sem=recv_sems.at[d - 1],
                device_id=(q,),
                device_id_type=pl.DeviceIdType.MESH,
            )
            rdma.start()
            rdmas.append(rdma)

        own = jnp.dot(
            a_ref[pl.ds(my * Mb, Mb), :],
            b_ref[:, :],
            preferred_element_type=jnp.float32,
        )

        acc = own
        for j in range(N_DEV - 1):
            rdmas[j].wait_recv()
            acc = acc + recv_buf[j, :, :].astype(jnp.float32)
        out_ref[:, :] = acc

        for rdma in rdmas:
            rdma.wait_send()

    return pl.pallas_call(
        body,
        out_shape=jax.ShapeDtypeStruct((Mb, N), jnp.float32),
        in_specs=[
            pl.BlockSpec(memory_space=pltpu.VMEM),
            pl.BlockSpec(memory_space=pltpu.VMEM),
        ],
        out_specs=pl.BlockSpec(memory_space=pltpu.VMEM),
        scratch_shapes=[
            pltpu.VMEM((N_DEV - 1, Mb, N), comm_dtype),
            pltpu.VMEM((N_DEV - 1, Mb, N), comm_dtype),
            pltpu.SemaphoreType.DMA((N_DEV - 1,)),
            pltpu.SemaphoreType.DMA((N_DEV - 1,)),
        ],
        compiler_params=pltpu.CompilerParams(collective_id=0),
    )(A, B)
